# Initial kernel scaffold; baseline (speedup 1.0000x reference)
import jax
import jax.numpy as jnp
from jax.experimental import pallas as pl


def kernel(X, L, mask, node_W, node_b, edge_W, edge_b, norm_nodes_gain, norm_nodes_bias, norm_edges_gain, norm_edges_bias):
    raise NotImplementedError("write your pallas kernel here")


# speedup vs baseline: 2.1595x; 2.1595x over previous
import math

import jax
import jax.numpy as jnp
from jax.experimental import pallas as pl

_K = 30
_QBLK = 128
_BIG = 3.0e38
_INTERPRET = False


def _nrm(x, axis=-1):
    n = jnp.sqrt(jnp.sum(x * x, axis=axis, keepdims=True))
    return x / jnp.maximum(n, 1e-12)


def _edge_kernel(xkt_ref, tfull_ref, tq_ref, wt_ref, b_ref, g_ref, bb_ref,
                 e_ref, eidx_ref):
    Q = tq_ref.shape[1]
    N = xkt_ref.shape[2]
    xk = xkt_ref[0]
    tq = tq_ref[0]
    tab = tfull_ref[0]
    wt = wt_ref[...]
    bias = b_ref[...]
    gain = g_ref[...]
    beta = bb_ref[...]

    lane = jax.lax.broadcasted_iota(jnp.int32, (Q, N), 1)
    d2 = None
    for c in range(3):
        d = tq[:, 9 + c:10 + c] - xk[c:c + 1, :]
        d2 = d * d if d2 is None else d2 + d * d
    D = jnp.sqrt(d2 + 1e-6)

    vals, idxs = [], []
    Dw = D
    for _ in range(_K):
        m = jnp.min(Dw, axis=1, keepdims=True)
        im = jnp.min(jnp.where(Dw == m, lane, N), axis=1, keepdims=True)
        vals.append(m)
        idxs.append(im)
        Dw = jnp.where(lane == im, _BIG, Dw)

    i_glob = (pl.program_id(1) * Q
              + jax.lax.broadcasted_iota(jnp.int32, (Q, 1), 0))
    i_glob_f = i_glob.astype(jnp.float32)
    freqs = jnp.exp(
        jax.lax.broadcasted_iota(jnp.int32, (1, 8), 1).astype(jnp.float32)
        * (2.0 * -(math.log(10000.0) / 16.0)))
    mus = (jax.lax.broadcasted_iota(jnp.int32, (1, 16), 1).astype(jnp.float32)
           * (20.0 / 15.0))

    oi = [tq[:, c:c + 1] for c in range(9)]
    xi = [tq[:, 9 + c:10 + c] for c in range(3)]

    for k in range(_K):
        val, idx = vals[k], idxs[k]
        ang = (idx.astype(jnp.float32) - i_glob_f) * freqs
        epos = jnp.concatenate([jnp.cos(ang), jnp.sin(ang)], 1)
        t = (val - mus) * (1.0 / 1.25)
        rbf = jnp.exp(-(t * t))
        oh = (lane == idx).astype(jnp.float32)
        gth = jnp.dot(oh, tab, preferred_element_type=jnp.float32,
                      precision=jax.lax.Precision.HIGHEST)
        oj = [gth[:, c:c + 1] for c in range(9)]
        dxn = [gth[:, 9 + c:10 + c] - xi[c] for c in range(3)]
        rb = lambda a: a.astype(jnp.bfloat16).astype(jnp.float32)
        oib = [rb(a) for a in oi]
        ojb = [rb(a) for a in oj]
        dxb = [rb(a) for a in dxn]
        du = [oib[3 * j] * dxb[0] + oib[3 * j + 1] * dxb[1]
              + oib[3 * j + 2] * dxb[2] for j in range(3)]
        dn = jnp.sqrt(du[0] * du[0] + du[1] * du[1] + du[2] * du[2])
        dn = jnp.maximum(dn, 1e-12)
        du = [u / dn for u in du]
        R = [[oib[c] * ojb[d] + oib[3 + c] * ojb[3 + d] + oib[6 + c] * ojb[6 + d]
              for d in range(3)] for c in range(3)]
        rxx, ryy, rzz = R[0][0], R[1][1], R[2][2]
        magx = 0.5 * jnp.sqrt(jnp.abs(1.0 + rxx - ryy - rzz))
        magy = 0.5 * jnp.sqrt(jnp.abs(1.0 - rxx + ryy - rzz))
        magz = 0.5 * jnp.sqrt(jnp.abs(1.0 - rxx - ryy + rzz))
        qx = jnp.sign(R[2][1] - R[1][2]) * magx
        qy = jnp.sign(R[0][2] - R[2][0]) * magy
        qz = jnp.sign(R[1][0] - R[0][1]) * magz
        qw = jnp.sqrt(jax.nn.relu(1.0 + rxx + ryy + rzz)) * 0.5
        qn = jnp.maximum(
            jnp.sqrt(qx * qx + qy * qy + qz * qz + qw * qw), 1e-12)
        qx, qy, qz, qw = qx / qn, qy / qn, qz / qn, qw / qn

        feat = jnp.concatenate(
            [epos, rbf, du[0], du[1], du[2], qx, qy, qz, qw], 1)
        h = jnp.dot(feat, wt, preferred_element_type=jnp.float32) + bias
        mu = jnp.mean(h, axis=1, keepdims=True)
        xc = h - mu
        var = jnp.sum(xc * xc, axis=1, keepdims=True) * (1.0 / 127.0)
        sig = jnp.sqrt(var + 1e-6)
        e_ref[0, :, k, :] = gain * xc / (sig + 1e-6) + beta

    eidx_ref[0] = jnp.concatenate(idxs, 1)


def _node_kernel(v_ref, wt_ref, b_ref, g_ref, bb_ref, o_ref):
    v = v_ref[0]
    h = jnp.dot(v, wt_ref[...], preferred_element_type=jnp.float32) + b_ref[...]
    mu = jnp.mean(h, axis=1, keepdims=True)
    xc = h - mu
    var = jnp.sum(xc * xc, axis=1, keepdims=True) * (1.0 / 127.0)
    sig = jnp.sqrt(var + 1e-6)
    o_ref[0] = g_ref[...] * xc / (sig + 1e-6) + bb_ref[...]


def _frames(xca):
    dx = xca[:, 1:, :] - xca[:, :-1, :]
    u = _nrm(dx)
    u2, u1 = u[:, :-2, :], u[:, 1:-1, :]
    n2 = _nrm(jnp.cross(u2, u1))
    o1 = _nrm(u2 - u1)
    o = jnp.stack([o1, n2, jnp.cross(o1, n2)], 2)
    o = o.reshape(o.shape[0], o.shape[1], 9)
    return jnp.pad(o, ((0, 0), (1, 2), (0, 0)))


def _dihedral_feats(x, eps=1e-7):
    b, n = x.shape[0], x.shape[1]
    xr = x[:, :, :3, :].reshape(b, 3 * n, 3)
    dx = xr[:, 1:, :] - xr[:, :-1, :]
    u = _nrm(dx)
    u2, u1, u0 = u[:, :-2, :], u[:, 1:-1, :], u[:, 2:, :]
    n2 = _nrm(jnp.cross(u2, u1))
    n1 = _nrm(jnp.cross(u1, u0))
    cosd = jnp.clip(jnp.sum(n2 * n1, -1), -1 + eps, 1 - eps)
    dang = jnp.sign(jnp.sum(u2 * n1, -1)) * jnp.arccos(cosd)
    dang = jnp.pad(dang, ((0, 0), (1, 2))).reshape(b, n, 3)
    return jnp.concatenate([jnp.cos(dang), jnp.sin(dang)], 2)


def kernel(X, L, mask, node_W, node_b, edge_W, edge_b,
           norm_nodes_gain, norm_nodes_bias, norm_edges_gain, norm_edges_bias):
    B, N = X.shape[0], X.shape[1]
    xca = X[:, :, 1, :]
    tab = jnp.concatenate([_frames(xca), xca], -1)
    xkt = jnp.swapaxes(xca, 1, 2)
    v6 = _dihedral_feats(X)

    r1 = lambda a: a.reshape(1, -1)
    E, E_idx = pl.pallas_call(
        _edge_kernel,
        grid=(B, N // _QBLK),
        in_specs=[
            pl.BlockSpec((1, 3, N), lambda b, i: (b, 0, 0)),
            pl.BlockSpec((1, N, 12), lambda b, i: (b, 0, 0)),
            pl.BlockSpec((1, _QBLK, 12), lambda b, i: (b, i, 0)),
            pl.BlockSpec((39, 128), lambda b, i: (0, 0)),
            pl.BlockSpec((1, 128), lambda b, i: (0, 0)),
            pl.BlockSpec((1, 128), lambda b, i: (0, 0)),
            pl.BlockSpec((1, 128), lambda b, i: (0, 0)),
        ],
        out_specs=[
            pl.BlockSpec((1, _QBLK, _K, 128), lambda b, i: (b, i, 0, 0)),
            pl.BlockSpec((1, _QBLK, _K), lambda b, i: (b, i, 0)),
        ],
        out_shape=[
            jax.ShapeDtypeStruct((B, N, _K, 128), jnp.float32),
            jax.ShapeDtypeStruct((B, N, _K), jnp.int32),
        ],
        interpret=_INTERPRET,
    )(xkt, tab, tab, edge_W.T, r1(edge_b), r1(norm_edges_gain),
      r1(norm_edges_bias))

    V = pl.pallas_call(
        _node_kernel,
        grid=(B, N // 512),
        in_specs=[
            pl.BlockSpec((1, 512, 6), lambda b, i: (b, i, 0)),
            pl.BlockSpec((6, 128), lambda b, i: (0, 0)),
            pl.BlockSpec((1, 128), lambda b, i: (0, 0)),
            pl.BlockSpec((1, 128), lambda b, i: (0, 0)),
            pl.BlockSpec((1, 128), lambda b, i: (0, 0)),
        ],
        out_specs=pl.BlockSpec((1, 512, 128), lambda b, i: (b, i, 0)),
        out_shape=jax.ShapeDtypeStruct((B, N, 128), jnp.float32),
        interpret=_INTERPRET,
    )(v6, node_W.T, r1(node_b), r1(norm_nodes_gain), r1(norm_nodes_bias))

    return V, E, E_idx
